# Initial kernel scaffold; baseline (speedup 1.0000x reference)
import jax
import jax.numpy as jnp
from jax.experimental import pallas as pl


def kernel(x, edge_attr, enc_W1, enc_b1, enc_W2, enc_b2, enc_W3, enc_b3, msg_W1, msg_b1, msg_W2, msg_b2, msg_W3, msg_b3, upd_W1, upd_b1, upd_W2, upd_b2, upd_W3, upd_b3, dec_W1, dec_b1, dec_W2, dec_b2, dec_W3, dec_b3, gn_gamma, gn_beta, edge_index):
    raise NotImplementedError("write your pallas kernel here")


# speedup vs baseline: 2.6347x; 2.6347x over previous
import functools

import jax
import jax.numpy as jnp
from jax import lax
from jax.experimental import pallas as pl
from jax.experimental.pallas import tpu as pltpu
from jax.experimental.pallas import tpu_sc as plsc

N = 10000
E = 320000
DX = 128
DE = 16
L = 16
H = 128
F = 1
NL = 4

NC = 2
NS = 16
NW = NC * NS
GRP = 128
NP = 10240
EP = 327680
EPW = EP // NW
CH = 1024
NG = CH // GRP
NCH = EPW // CH
RS = NP // NS


def _elu(x):
    z = jnp.where(x > 0, 0.0, x)
    e = jnp.exp(z)
    h = 0.5 * z
    r = jnp.tanh(h) * (1.0 + e)
    r = jnp.where(h == 0.0, z, r)
    return jnp.where(x > 0, x, r)


def _dot(a, b):
    return jnp.dot(a.astype(jnp.bfloat16), b.astype(jnp.bfloat16),
                   preferred_element_type=jnp.float32)


def _gather_body(h_hbm, src_hbm, dst_hbm, hs_hbm, hd_hbm,
                 sidx, didx, srows, drows, sem):
    cid = lax.axis_index("c")
    sid = lax.axis_index("s")
    wid = sid * NC + cid

    def chunk(ci, carry):
        grp0 = wid * (EPW // GRP) + ci * NG
        row0 = wid * EPW + ci * CH
        pltpu.sync_copy(src_hbm.at[pl.ds(grp0, NG)], sidx)
        pltpu.sync_copy(dst_hbm.at[pl.ds(grp0, NG)], didx)
        descs = []
        for j in range(NG):
            descs.append(pltpu.async_copy(
                h_hbm.at[sidx.at[j]], srows.at[pl.ds(j * GRP, GRP)], sem))
            descs.append(pltpu.async_copy(
                h_hbm.at[didx.at[j]], drows.at[pl.ds(j * GRP, GRP)], sem))
        for d in descs:
            d.wait()
        pltpu.sync_copy(srows, hs_hbm.at[pl.ds(row0, CH)])
        pltpu.sync_copy(drows, hd_hbm.at[pl.ds(row0, CH)])
        return carry

    lax.fori_loop(0, NCH, chunk, 0)


@functools.cache
def _make_gather():
    return pl.kernel(
        _gather_body,
        out_type=(jax.ShapeDtypeStruct((EP, L), jnp.float32),
                  jax.ShapeDtypeStruct((EP, L), jnp.float32)),
        mesh=plsc.VectorSubcoreMesh(core_axis_name="c", subcore_axis_name="s",
                                    num_cores=NC, num_subcores=NS),
        scratch_types=[
            pltpu.VMEM((NG, GRP), jnp.int32),
            pltpu.VMEM((NG, GRP), jnp.int32),
            pltpu.VMEM((CH, L), jnp.float32),
            pltpu.VMEM((CH, L), jnp.float32),
            pltpu.SemaphoreType.DMA,
        ],
        compiler_params=pltpu.CompilerParams(use_tc_tiling_on_sc=False),
    )


def _gather_call(h, src2d, dst2d):
    return _make_gather()(h, src2d, dst2d)


def _scatter_body(m_hbm, dst_hbm, out_hbm, didx, mbuf, zbuf, acc):
    cid = lax.axis_index("c")
    sid = lax.axis_index("s")
    wid = sid * NC + cid

    def zrow(i, carry):
        zbuf[i, :] = jnp.zeros((L,), jnp.float32)
        return carry

    lax.fori_loop(0, RS, zrow, 0)
    pltpu.sync_copy(zbuf, acc.at[pl.ds(sid * RS, RS)])
    plsc.subcore_barrier()

    def chunk(ci, carry):
        grp0 = wid * (EPW // GRP) + ci * NG
        row0 = wid * EPW + ci * CH
        pltpu.sync_copy(m_hbm.at[pl.ds(row0, CH)], mbuf)
        pltpu.sync_copy(dst_hbm.at[pl.ds(grp0, NG)], didx)
        for j in range(NG):
            pltpu.sync_copy(mbuf.at[pl.ds(j * GRP, GRP)],
                            acc.at[didx.at[j]], add=True)
        return carry

    lax.fori_loop(0, NCH, chunk, 0)
    plsc.subcore_barrier()
    pltpu.sync_copy(acc.at[pl.ds(sid * RS, RS)], zbuf)
    pltpu.sync_copy(zbuf, out_hbm.at[cid, pl.ds(sid * RS, RS)])


@functools.cache
def _make_scatter():
    return pl.kernel(
        _scatter_body,
        out_type=jax.ShapeDtypeStruct((NC, NP, L), jnp.float32),
        mesh=plsc.VectorSubcoreMesh(core_axis_name="c", subcore_axis_name="s",
                                    num_cores=NC, num_subcores=NS),
        scratch_types=[
            pltpu.VMEM((NG, GRP), jnp.int32),
            pltpu.VMEM((CH, L), jnp.float32),
            pltpu.VMEM((RS, L), jnp.float32),
            pltpu.VMEM_SHARED((NP, L), jnp.float32),
        ],
        compiler_params=pltpu.CompilerParams(use_tc_tiling_on_sc=False),
    )


def _scatter_call(m, dst2d):
    return _make_scatter()(m, dst2d)


BN = 1024
BE = 2048


def _enc_dec_body(x_ref, W1, b1, W2, b2, W3, b3, o_ref, *, outer_elu):
    h = _dot(x_ref[...], W1[...]) + b1[...]
    h = _elu(h)
    h = _dot(h, W2[...]) + b2[...]
    h = _elu(h)
    h = _dot(h, W3[...]) + b3[...]
    o_ref[...] = _elu(h) if outer_elu else h


def _mlp3_call(x, W1, b1, W2, b2, W3, b3, outer_elu):
    n = x.shape[0]
    dout = W3.shape[1]
    full = lambda *s: pl.BlockSpec(s, lambda i: (0,) * len(s))
    return pl.pallas_call(
        functools.partial(_enc_dec_body, outer_elu=outer_elu),
        grid=(n // BN,),
        in_specs=[
            pl.BlockSpec((BN, x.shape[1]), lambda i: (i, 0)),
            full(*W1.shape), full(*b1.shape),
            full(*W2.shape), full(*b2.shape),
            full(*W3.shape), full(*b3.shape),
        ],
        out_specs=pl.BlockSpec((BN, dout), lambda i: (i, 0)),
        out_shape=jax.ShapeDtypeStruct((n, dout), jnp.float32),
    )(x, W1, b1, W2, b2, W3, b3)


def _msg_body(hs_ref, hd_ref, ea_ref, W1, b1, W2, b2, W3, b3, o_ref):
    cat = jnp.concatenate([hs_ref[...], hd_ref[...], ea_ref[...]], axis=1)
    h = _dot(cat, W1[...]) + b1[...]
    h = _elu(h)
    h = _dot(h, W2[...]) + b2[...]
    h = _elu(h)
    o_ref[...] = _dot(h, W3[...]) + b3[...]


def _msg_call(hs, hd, ea, W1, b1, W2, b2, W3, b3):
    full = lambda *s: pl.BlockSpec(s, lambda i: (0,) * len(s))
    return pl.pallas_call(
        _msg_body,
        grid=(EP // BE,),
        in_specs=[
            pl.BlockSpec((BE, L), lambda i: (i, 0)),
            pl.BlockSpec((BE, L), lambda i: (i, 0)),
            pl.BlockSpec((BE, DE), lambda i: (i, 0)),
            full(*W1.shape), full(*b1.shape),
            full(*W2.shape), full(*b2.shape), full(*W3.shape), full(*b3.shape),
        ],
        out_specs=pl.BlockSpec((BE, L), lambda i: (i, 0)),
        out_shape=jax.ShapeDtypeStruct((EP, L), jnp.float32),
    )(hs, hd, ea, W1, b1, W2, b2, W3, b3)


def _upd_body(h_ref, agg_ref, U1, b1, W2, b2, W3, b3, gam, bet, o_ref):
    hcur = h_ref[...]
    agg = agg_ref[0] + agg_ref[1]
    cat = jnp.concatenate([hcur, agg], axis=1)
    u = _dot(cat, U1[...]) + b1[...]
    u = _elu(u)
    u = _dot(u, W2[...]) + b2[...]
    u = _elu(u)
    u = _dot(u, W3[...]) + b3[...]
    u = _elu(u)
    def tree8(a):
        b = [a[i] + a[i + 4] for i in range(4)]
        c = [b[0] + b[2], b[1] + b[3]]
        return c[0] + c[1]

    col = lax.broadcasted_iota(jnp.int32, u.shape, 1)
    m0 = col < (L // 2)
    ucols = [u[:, i:i + 1] for i in range(L)]
    s0 = tree8(ucols[:8])
    s1 = tree8(ucols[8:])
    mu = jnp.where(m0, s0, s1) / (L // 2)
    d = u - mu
    dcols = [d[:, i:i + 1] for i in range(L)]
    q0 = tree8([c * c for c in dcols[:8]])
    q1 = tree8([c * c for c in dcols[8:]])
    var = jnp.where(m0, q0, q1) / (L // 2)
    o_ref[...] = d / jnp.sqrt(var + 1e-5) * gam[...] + bet[...]


def _upd_call(h, agg2, U1, b1, W2, b2, W3, b3, gam, bet):
    full = lambda *s: pl.BlockSpec(s, lambda i: (0,) * len(s))
    return pl.pallas_call(
        _upd_body,
        grid=(NP // BN,),
        in_specs=[
            pl.BlockSpec((BN, L), lambda i: (i, 0)),
            pl.BlockSpec((NC, BN, L), lambda i: (0, i, 0)),
            full(*U1.shape), full(*b1.shape),
            full(*W2.shape), full(*b2.shape), full(*W3.shape), full(*b3.shape),
            full(*gam.shape), full(*bet.shape),
        ],
        out_specs=pl.BlockSpec((BN, L), lambda i: (i, 0)),
        out_shape=jax.ShapeDtypeStruct((NP, L), jnp.float32),
    )(h, agg2, U1, b1, W2, b2, W3, b3, gam, bet)


def kernel(x, edge_attr, enc_W1, enc_b1, enc_W2, enc_b2, enc_W3, enc_b3,
           msg_W1, msg_b1, msg_W2, msg_b2, msg_W3, msg_b3,
           upd_W1, upd_b1, upd_W2, upd_b2, upd_W3, upd_b3,
           dec_W1, dec_b1, dec_W2, dec_b2, dec_W3, dec_b3,
           gn_gamma, gn_beta, edge_index):
    r2 = lambda v: v.reshape(1, -1)

    xp = jnp.zeros((NP, DX), jnp.float32).at[:N].set(x)
    eap = jnp.zeros((EP, DE), jnp.float32).at[:E].set(edge_attr)
    src = jnp.zeros((EP,), jnp.int32).at[:E].set(edge_index[0])
    dst = jnp.full((EP,), N, jnp.int32).at[:E].set(edge_index[1])
    src2d = src.reshape(EP // GRP, GRP)
    dst2d = dst.reshape(EP // GRP, GRP)

    h = _mlp3_call(xp, enc_W1, r2(enc_b1), enc_W2, r2(enc_b2),
                   enc_W3, r2(enc_b3), outer_elu=True)

    for l in range(NL):
        hs, hd = _gather_call(h, src2d, dst2d)
        m = _msg_call(hs, hd, eap, msg_W1[l],
                      r2(msg_b1[l]), msg_W2[l], r2(msg_b2[l]),
                      msg_W3[l], r2(msg_b3[l]))
        agg2 = _scatter_call(m, dst2d)
        h = _upd_call(h, agg2,
                      upd_W1[l], r2(upd_b1[l]),
                      upd_W2[l], r2(upd_b2[l]), upd_W3[l], r2(upd_b3[l]),
                      r2(gn_gamma), r2(gn_beta))

    y = _mlp3_call(h, dec_W1, r2(dec_b1), dec_W2, r2(dec_b2),
                   dec_W3, r2(dec_b3), outer_elu=False)
    return y[:N]
